# Initial kernel scaffold; baseline (speedup 1.0000x reference)
import jax
import jax.numpy as jnp
from jax.experimental import pallas as pl


def kernel(boxes, scores):
    raise NotImplementedError("write your pallas kernel here")


# speedup vs baseline: 176.0334x; 176.0334x over previous
import jax
import jax.numpy as jnp
from jax.experimental import pallas as pl
from jax.experimental.pallas import tpu as pltpu

K = 2000
KPAD = 2048
IOU_THRESH = 0.7
MAX_OUT = 300


def _nms_kernel(bc_ref, br_ref, keep_ref, m_ref):
    x1c = bc_ref[:, 0:1]
    y1c = bc_ref[:, 1:2]
    x2c = bc_ref[:, 2:3]
    y2c = bc_ref[:, 3:4]
    x1r = br_ref[0:1, :]
    y1r = br_ref[1:2, :]
    x2r = br_ref[2:3, :]
    y2r = br_ref[3:4, :]

    area_c = (x2c - x1c) * (y2c - y1c)
    area_r = (x2r - x1r) * (y2r - y1r)
    xx1 = jnp.maximum(x1c, x1r)
    yy1 = jnp.maximum(y1c, y1r)
    xx2 = jnp.minimum(x2c, x2r)
    yy2 = jnp.minimum(y2c, y2r)
    iw = jnp.clip(xx2 - xx1, 0.0)
    ih = jnp.clip(yy2 - yy1, 0.0)
    inter = iw * ih
    union = area_c + area_r - inter
    iou = inter / jnp.maximum(union, 1e-9)

    gi = jax.lax.broadcasted_iota(jnp.int32, (KPAD, KPAD), 0)
    gj = jax.lax.broadcasted_iota(jnp.int32, (KPAD, KPAD), 1)
    m_ref[:, :] = ((iou > IOU_THRESH) & (gj > gi)).astype(jnp.bfloat16)

    def cond(c):
        return c[1]

    def body(c):
        k, _ = c
        cnt = jax.lax.dot_general(
            k.astype(jnp.bfloat16), m_ref[:, :],
            (((1,), (0,)), ((), ())),
            preferred_element_type=jnp.float32,
        )
        k_new = (cnt == 0.0).astype(jnp.float32)
        return k_new, jnp.any(k_new != k)

    k0 = jnp.ones((1, KPAD), jnp.float32)
    k_final, _ = jax.lax.while_loop(cond, body, (k0, True))
    keep_ref[:, :] = k_final


def _nms_keep(boxes_p, boxesT):
    return pl.pallas_call(
        _nms_kernel,
        out_shape=jax.ShapeDtypeStruct((1, KPAD), jnp.float32),
        scratch_shapes=[pltpu.VMEM((KPAD, KPAD), jnp.bfloat16)],
    )(boxes_p, boxesT)


def kernel(boxes, scores):
    top_scores, order = jax.lax.top_k(scores, K)
    top_boxes = jnp.take(boxes, order, axis=0)
    boxes_p = jnp.concatenate(
        [top_boxes, jnp.zeros((KPAD - K, 4), jnp.float32)], axis=0
    )
    keep = _nms_keep(boxes_p, boxes_p.T)
    keep_b = keep[0, :K] > 0.5
    masked = jnp.where(keep_b, top_scores, -1e9)
    final_scores, final_idx = jax.lax.top_k(masked, MAX_OUT)
    final_boxes = jnp.take(top_boxes, final_idx, axis=0)
    return final_boxes, final_scores
